# Initial kernel scaffold; baseline (speedup 1.0000x reference)
import functools

import jax
import jax.numpy as jnp
from jax import lax
from jax.experimental import pallas as pl
from jax.experimental.pallas import tpu as pltpu
from jax.experimental.pallas import tpu_sc as plsc

N = 10000
E = 320000
D = 128

NC = 2
NS = 16
NW = NC * NS
L = 16

NP = 10240
EPW = E // NW
CH = 80
NCH = EPW // CH
RPT = NP // NS
NZC = RPT // CH

_MESH = dict(core_axis_name="c", subcore_axis_name="s", num_cores=NC,
             num_subcores=NS)


def _sc_degree(col):
  mesh = plsc.VectorSubcoreMesh(**_MESH)

  @functools.partial(
      pl.kernel,
      out_type=jax.ShapeDtypeStruct((NW, NP), jnp.float32),
      mesh=mesh,
      scratch_types=[
          pltpu.VMEM((NP,), jnp.float32),
          pltpu.VMEM((CH,), jnp.int32),
      ],
  )
  def k(col_hbm, out_hbm, deg_v, cbuf):
    cid = lax.axis_index("c")
    sid = lax.axis_index("s")
    wid = sid * NC + cid
    ones = jnp.ones((L,), jnp.float32)

    def zbody(r, carry):
      deg_v[pl.ds(r * L, L)] = jnp.zeros((L,), jnp.float32)
      return carry

    lax.fori_loop(0, NP // L, zbody, 0)

    ebase = wid * EPW

    def cbody(i, carry):
      pltpu.sync_copy(col_hbm.at[pl.ds(ebase + i * CH, CH)], cbuf)
      for j in range(CH // L):
        idx = cbuf[pl.ds(j * L, L)]
        plsc.addupdate_scatter(deg_v, [idx], ones)
      return carry

    lax.fori_loop(0, NCH, cbody, 0)
    pltpu.sync_copy(deg_v, out_hbm.at[wid])

  return k(col)


def _sc_edge_scatter(ht, row, col):
  mesh = plsc.VectorSubcoreMesh(**_MESH)

  @functools.partial(
      pl.kernel,
      out_type=jax.ShapeDtypeStruct((NC, NP, D), jnp.float32),
      mesh=mesh,
      scratch_types=[
          pltpu.VMEM((CH,), jnp.int32),
          pltpu.VMEM((CH,), jnp.int32),
          pltpu.VMEM((CH, D), jnp.float32),
          pltpu.VMEM((CH, D), jnp.float32),
          pltpu.VMEM_SHARED((NP, D), jnp.float32),
          pltpu.SemaphoreType.DMA,
      ],
  )
  def k(ht_hbm, row_hbm, col_hbm, out_hbm, rbuf, cbuf, rows_v, stage_v,
        acc_sh, sem):
    cid = lax.axis_index("c")
    sid = lax.axis_index("s")
    wid = sid * NC + cid

    def zbody(r, carry):
      for j in range(D // L):
        stage_v[r, pl.ds(j * L, L)] = jnp.zeros((L,), jnp.float32)
      return carry

    lax.fori_loop(0, CH, zbody, 0)
    for j in range(NZC):
      pltpu.sync_copy(stage_v, acc_sh.at[pl.ds(sid * RPT + j * CH, CH)])
    plsc.subcore_barrier()

    ebase = wid * EPW

    def cbody(i, carry):
      off = ebase + i * CH
      pltpu.sync_copy(row_hbm.at[pl.ds(off, CH)], rbuf)
      pltpu.sync_copy(col_hbm.at[pl.ds(off, CH)], cbuf)
      pltpu.async_copy(ht_hbm.at[rbuf], rows_v, sem).wait()
      pltpu.sync_copy(rows_v, acc_sh.at[cbuf], add=True)
      return carry

    lax.fori_loop(0, NCH, cbody, 0)
    plsc.subcore_barrier()

    for j in range(NZC):
      base = sid * RPT + j * CH
      pltpu.sync_copy(acc_sh.at[pl.ds(base, CH)], stage_v)
      pltpu.sync_copy(stage_v, out_hbm.at[cid, pl.ds(base, CH)])

  return k(ht, row, col)


_RB = 1024
_GRID = NP // _RB


def _tc1_body(degT_ref, x_ref, w_ref, ht_ref, dis_ref):
  deg = jnp.sum(degT_ref[...], axis=1, keepdims=True)
  dis = jnp.where(deg > 0.0, lax.rsqrt(deg), 0.0)
  p = jnp.dot(x_ref[...], w_ref[...], preferred_element_type=jnp.float32)
  ht_ref[...] = dis * p
  dis_ref[...] = dis


def _tc_layer1(degT, xp, W1):
  return pl.pallas_call(
      _tc1_body,
      grid=(_GRID,),
      in_specs=[
          pl.BlockSpec((_RB, NW), lambda i: (i, 0)),
          pl.BlockSpec((_RB, D), lambda i: (i, 0)),
          pl.BlockSpec((D, D), lambda i: (0, 0)),
      ],
      out_specs=[
          pl.BlockSpec((_RB, D), lambda i: (i, 0)),
          pl.BlockSpec((_RB, 1), lambda i: (i, 0)),
      ],
      out_shape=[
          jax.ShapeDtypeStruct((NP, D), jnp.float32),
          jax.ShapeDtypeStruct((NP, 1), jnp.float32),
      ],
  )(degT, xp, W1)


def _tc2_body(a_ref, b_ref, dis_ref, bias_ref, w_ref, out_ref, ht_ref):
  dis = dis_ref[...]
  out = dis * (a_ref[...] + b_ref[...]) + bias_ref[...]
  out_ref[...] = out
  ht_ref[...] = dis * jnp.dot(out, w_ref[...],
                              preferred_element_type=jnp.float32)


def _tc_layer2(accA, accB, dis, bias, W2):
  return pl.pallas_call(
      _tc2_body,
      grid=(_GRID,),
      in_specs=[
          pl.BlockSpec((_RB, D), lambda i: (i, 0)),
          pl.BlockSpec((_RB, D), lambda i: (i, 0)),
          pl.BlockSpec((_RB, 1), lambda i: (i, 0)),
          pl.BlockSpec((1, D), lambda i: (0, 0)),
          pl.BlockSpec((D, D), lambda i: (0, 0)),
      ],
      out_specs=[
          pl.BlockSpec((_RB, D), lambda i: (i, 0)),
          pl.BlockSpec((_RB, D), lambda i: (i, 0)),
      ],
      out_shape=[
          jax.ShapeDtypeStruct((NP, D), jnp.float32),
          jax.ShapeDtypeStruct((NP, D), jnp.float32),
      ],
  )(accA, accB, dis, bias, W2)


def _tc3_body(a_ref, b_ref, dis_ref, bias_ref, out_ref):
  out_ref[...] = dis_ref[...] * (a_ref[...] + b_ref[...]) + bias_ref[...]


def _tc_layer3(accA, accB, dis, bias):
  return pl.pallas_call(
      _tc3_body,
      grid=(_GRID,),
      in_specs=[
          pl.BlockSpec((_RB, D), lambda i: (i, 0)),
          pl.BlockSpec((_RB, D), lambda i: (i, 0)),
          pl.BlockSpec((_RB, 1), lambda i: (i, 0)),
          pl.BlockSpec((1, D), lambda i: (0, 0)),
      ],
      out_specs=pl.BlockSpec((_RB, D), lambda i: (i, 0)),
      out_shape=jax.ShapeDtypeStruct((NP, D), jnp.float32),
  )(accA, accB, dis, bias)


def kernel(x, edge_index, W1, b1, W2, b2):
  row = edge_index[0]
  col = edge_index[1]
  xp = jnp.pad(x, ((0, NP - N), (0, 0)))

  deg_parts = _sc_degree(col)
  ht1, dis = _tc_layer1(deg_parts.T, xp, W1)
  acc1 = _sc_edge_scatter(ht1, row, col)
  out1p, ht2 = _tc_layer2(acc1[0], acc1[1], dis, b1.reshape(1, D), W2)
  acc2 = _sc_edge_scatter(ht2, row, col)
  out2p = _tc_layer3(acc2[0], acc2[1], dis, b2.reshape(1, D))

  out1 = out1p[:N]
  out2 = out2p[:N]
  return (out2, (x, out1, out2))


# speedup vs baseline: 9.0201x; 9.0201x over previous
import functools

import jax
import jax.numpy as jnp
from jax import lax
from jax.experimental import pallas as pl
from jax.experimental.pallas import tpu as pltpu
from jax.experimental.pallas import tpu_sc as plsc

N = 10000
E = 320000
D = 128

NC = 2
NS = 16
NW = NC * NS
L = 16

NP = 10240
EPW = E // NW
CH = 80
NCH = EPW // CH
RPT = NP // NS
NZC = RPT // CH

_MESH = dict(core_axis_name="c", subcore_axis_name="s", num_cores=NC,
             num_subcores=NS)


def _sc_degree(col):
  mesh = plsc.VectorSubcoreMesh(**_MESH)

  @functools.partial(
      pl.kernel,
      out_type=jax.ShapeDtypeStruct((NC, NP, D), jnp.float32),
      mesh=mesh,
      scratch_types=[
          pltpu.VMEM((CH, D), jnp.float32),
          pltpu.VMEM((CH, D), jnp.float32),
          pltpu.VMEM((CH,), jnp.int32),
          pltpu.VMEM_SHARED((NP, D), jnp.float32),
      ],
  )
  def k(col_hbm, out_hbm, ones_v, zeros_v, cbuf, deg_sh):
    cid = lax.axis_index("c")
    sid = lax.axis_index("s")
    wid = sid * NC + cid

    def fill(r, carry):
      for j in range(D // L):
        ones_v[r, pl.ds(j * L, L)] = jnp.ones((L,), jnp.float32)
        zeros_v[r, pl.ds(j * L, L)] = jnp.zeros((L,), jnp.float32)
      return carry

    lax.fori_loop(0, CH, fill, 0)
    for j in range(NZC):
      pltpu.sync_copy(zeros_v, deg_sh.at[pl.ds(sid * RPT + j * CH, CH)])
    plsc.subcore_barrier()

    ebase = wid * EPW

    def cbody(i, carry):
      pltpu.sync_copy(col_hbm.at[pl.ds(ebase + i * CH, CH)], cbuf)
      pltpu.sync_copy(ones_v, deg_sh.at[cbuf], add=True)
      return carry

    lax.fori_loop(0, NCH, cbody, 0)
    plsc.subcore_barrier()
    for j in range(NZC):
      base = sid * RPT + j * CH
      pltpu.sync_copy(deg_sh.at[pl.ds(base, CH)], zeros_v)
      pltpu.sync_copy(zeros_v, out_hbm.at[cid, pl.ds(base, CH)])

  return k(col)


def _sc_edge_scatter(ht, row, col):
  mesh = plsc.VectorSubcoreMesh(**_MESH)

  @functools.partial(
      pl.kernel,
      out_type=jax.ShapeDtypeStruct((NC, NP, D), jnp.float32),
      mesh=mesh,
      scratch_types=[
          pltpu.VMEM((CH,), jnp.int32),
          pltpu.VMEM((CH,), jnp.int32),
          pltpu.VMEM((CH, D), jnp.float32),
          pltpu.VMEM((CH, D), jnp.float32),
          pltpu.VMEM_SHARED((NP, D), jnp.float32),
          pltpu.SemaphoreType.DMA,
      ],
  )
  def k(ht_hbm, row_hbm, col_hbm, out_hbm, rbuf, cbuf, rows_v, stage_v,
        acc_sh, sem):
    cid = lax.axis_index("c")
    sid = lax.axis_index("s")
    wid = sid * NC + cid

    def zbody(r, carry):
      for j in range(D // L):
        stage_v[r, pl.ds(j * L, L)] = jnp.zeros((L,), jnp.float32)
      return carry

    lax.fori_loop(0, CH, zbody, 0)
    for j in range(NZC):
      pltpu.sync_copy(stage_v, acc_sh.at[pl.ds(sid * RPT + j * CH, CH)])
    plsc.subcore_barrier()

    ebase = wid * EPW

    def cbody(i, carry):
      off = ebase + i * CH
      pltpu.sync_copy(row_hbm.at[pl.ds(off, CH)], rbuf)
      pltpu.sync_copy(col_hbm.at[pl.ds(off, CH)], cbuf)
      pltpu.async_copy(ht_hbm.at[rbuf], rows_v, sem).wait()
      pltpu.sync_copy(rows_v, acc_sh.at[cbuf], add=True)
      return carry

    lax.fori_loop(0, NCH, cbody, 0)
    plsc.subcore_barrier()

    for j in range(NZC):
      base = sid * RPT + j * CH
      pltpu.sync_copy(acc_sh.at[pl.ds(base, CH)], stage_v)
      pltpu.sync_copy(stage_v, out_hbm.at[cid, pl.ds(base, CH)])

  return k(ht, row, col)


_RB = 1024
_GRID = NP // _RB


def _tc1_body(deg0_ref, deg1_ref, x_ref, w_ref, ht_ref, dis_ref):
  deg = (deg0_ref[...] + deg1_ref[...])[:, 0:1]
  dis = jnp.where(deg > 0.0, lax.rsqrt(deg), 0.0)
  p = jnp.dot(x_ref[...], w_ref[...], preferred_element_type=jnp.float32)
  ht_ref[...] = dis * p
  dis_ref[...] = dis


def _tc_layer1(deg0, deg1, xp, W1):
  return pl.pallas_call(
      _tc1_body,
      grid=(_GRID,),
      in_specs=[
          pl.BlockSpec((_RB, D), lambda i: (i, 0)),
          pl.BlockSpec((_RB, D), lambda i: (i, 0)),
          pl.BlockSpec((_RB, D), lambda i: (i, 0)),
          pl.BlockSpec((D, D), lambda i: (0, 0)),
      ],
      out_specs=[
          pl.BlockSpec((_RB, D), lambda i: (i, 0)),
          pl.BlockSpec((_RB, 1), lambda i: (i, 0)),
      ],
      out_shape=[
          jax.ShapeDtypeStruct((NP, D), jnp.float32),
          jax.ShapeDtypeStruct((NP, 1), jnp.float32),
      ],
  )(deg0, deg1, xp, W1)


def _tc2_body(a_ref, b_ref, dis_ref, bias_ref, w_ref, out_ref, ht_ref):
  dis = dis_ref[...]
  out = dis * (a_ref[...] + b_ref[...]) + bias_ref[...]
  out_ref[...] = out
  ht_ref[...] = dis * jnp.dot(out, w_ref[...],
                              preferred_element_type=jnp.float32)


def _tc_layer2(accA, accB, dis, bias, W2):
  return pl.pallas_call(
      _tc2_body,
      grid=(_GRID,),
      in_specs=[
          pl.BlockSpec((_RB, D), lambda i: (i, 0)),
          pl.BlockSpec((_RB, D), lambda i: (i, 0)),
          pl.BlockSpec((_RB, 1), lambda i: (i, 0)),
          pl.BlockSpec((1, D), lambda i: (0, 0)),
          pl.BlockSpec((D, D), lambda i: (0, 0)),
      ],
      out_specs=[
          pl.BlockSpec((_RB, D), lambda i: (i, 0)),
          pl.BlockSpec((_RB, D), lambda i: (i, 0)),
      ],
      out_shape=[
          jax.ShapeDtypeStruct((NP, D), jnp.float32),
          jax.ShapeDtypeStruct((NP, D), jnp.float32),
      ],
  )(accA, accB, dis, bias, W2)


def _tc3_body(a_ref, b_ref, dis_ref, bias_ref, out_ref):
  out_ref[...] = dis_ref[...] * (a_ref[...] + b_ref[...]) + bias_ref[...]


def _tc_layer3(accA, accB, dis, bias):
  return pl.pallas_call(
      _tc3_body,
      grid=(_GRID,),
      in_specs=[
          pl.BlockSpec((_RB, D), lambda i: (i, 0)),
          pl.BlockSpec((_RB, D), lambda i: (i, 0)),
          pl.BlockSpec((_RB, 1), lambda i: (i, 0)),
          pl.BlockSpec((1, D), lambda i: (0, 0)),
      ],
      out_specs=pl.BlockSpec((_RB, D), lambda i: (i, 0)),
      out_shape=jax.ShapeDtypeStruct((NP, D), jnp.float32),
  )(accA, accB, dis, bias)


def kernel(x, edge_index, W1, b1, W2, b2):
  row = edge_index[0]
  col = edge_index[1]
  xp = jnp.pad(x, ((0, NP - N), (0, 0)))

  deg_parts = _sc_degree(col)
  ht1, dis = _tc_layer1(deg_parts[0], deg_parts[1], xp, W1)
  acc1 = _sc_edge_scatter(ht1, row, col)
  out1p, ht2 = _tc_layer2(acc1[0], acc1[1], dis, b1.reshape(1, D), W2)
  acc2 = _sc_edge_scatter(ht2, row, col)
  out2p = _tc_layer3(acc2[0], acc2[1], dis, b2.reshape(1, D))

  out1 = out1p[:N]
  out2 = out2p[:N]
  return (out2, (x, out1, out2))
